# baseline (speedup 1.0000x reference)
import functools

import jax
import jax.numpy as jnp
from jax import lax
from jax.experimental import pallas as pl
from jax.experimental.pallas import tpu as pltpu
from jax.experimental.pallas import tpu_sc as plsc

N = 10000
E = 320000
D = 128
NC = 2
NS = 16
NPC = N // NC
ACC = 5120
CH = 80
NCH_H = 128
NHALF = 2
NCHUNK = NHALF * NCH_H
EPT = NCHUNK * CH
EPAD = NS * EPT
NW = NC * NS
EPW = E // NW
DCHUNK = EPW // CH
ZRPT = ACC // NS
RCH = 80
OCH = 40
ORPT = 320

_mesh = plsc.VectorSubcoreMesh(core_axis_name="c", subcore_axis_name="s")


@functools.partial(
    pl.kernel,
    out_type=jax.ShapeDtypeStruct((NC * N,), jnp.float32),
    mesh=_mesh,
    scratch_types=[
        pltpu.VMEM((DCHUNK, CH), jnp.int32),
        pltpu.VMEM((CH,), jnp.float32),
        pltpu.VMEM((2000,), jnp.float32),
        pltpu.VMEM_SHARED((N,), jnp.float32),
        pltpu.SemaphoreType.DMA,
    ],
)
def _sc_degree(dst_hbm, cnt_hbm, dst_v, ones_v, zbuf_v, deg_sh, sem):
    c = lax.axis_index("c")
    s = lax.axis_index("s")
    wid = c * NS + s

    def zb(i, carry):
        zbuf_v[pl.ds(i * 16, 16)] = jnp.zeros((16,), jnp.float32)
        return carry
    lax.fori_loop(0, 125, zb, 0)
    @pl.when(s < 5)
    def _zero_acc():
        pltpu.sync_copy(zbuf_v, deg_sh.at[pl.ds(s * 2000, 2000)])

    for k in range(CH // 16):
        ones_v[pl.ds(k * 16, 16)] = jnp.ones((16,), jnp.float32)
    pltpu.sync_copy(dst_hbm.at[wid], dst_v)
    plsc.subcore_barrier()

    def body(j, carry):
        pltpu.async_copy(ones_v, deg_sh.at[dst_v.at[j]], sem, add=True).wait()
        return carry
    lax.fori_loop(0, DCHUNK, body, 0)
    plsc.subcore_barrier()

    @pl.when(s < 5)
    def _copy_out():
        pltpu.sync_copy(deg_sh.at[pl.ds(s * 2000, 2000)], zbuf_v)
        pltpu.sync_copy(zbuf_v, cnt_hbm.at[pl.ds(c * N + s * 2000, 2000)])


@functools.partial(
    pl.kernel,
    out_type=jax.ShapeDtypeStruct((N, D), jnp.float32),
    mesh=_mesh,
    scratch_types=[
        pltpu.VMEM((NCH_H, CH), jnp.int32),
        pltpu.VMEM((NCH_H, CH), jnp.int32),
        pltpu.VMEM((4, CH, D), jnp.float32),
        pltpu.VMEM_SHARED((ACC, D), jnp.float32),
        pltpu.SemaphoreType.DMA,
        pltpu.SemaphoreType.DMA,
        pltpu.SemaphoreType.DMA,
        pltpu.SemaphoreType.DMA,
        pltpu.SemaphoreType.DMA,
        pltpu.SemaphoreType.DMA,
        pltpu.SemaphoreType.DMA,
        pltpu.SemaphoreType.DMA,
    ],
)
def _sc_scatter(g_hbm, src_hbm, dst_hbm, out_hbm,
                src_v, dst_v, rows_v, acc_sh,
                gsem0, gsem1, gsem2, gsem3, ssem0, ssem1, ssem2, ssem3):
    c = lax.axis_index("c")
    s = lax.axis_index("s")

    def zr(r, carry):
        def zc(k, carry2):
            rows_v[0, r, pl.ds(k * 16, 16)] = jnp.zeros((16,), jnp.float32)
            return carry2
        return lax.fori_loop(0, D // 16, zc, carry)
    lax.fori_loop(0, RCH, zr, 0)

    def zchunk(k, carry):
        pltpu.sync_copy(rows_v.at[0], acc_sh.at[pl.ds(s * ZRPT + k * RCH, RCH)])
        return carry
    lax.fori_loop(0, ZRPT // RCH, zchunk, 0)

    gsems = (gsem0, gsem1, gsem2, gsem3)
    ssems = (ssem0, ssem1, ssem2, ssem3)
    lo = c * NPC

    def _gwait(b):
        pltpu.make_async_copy(g_hbm.at[src_v.at[0]], rows_v.at[b],
                              gsems[b]).wait()

    def _swait(b):
        pltpu.make_async_copy(rows_v.at[b], acc_sh.at[dst_v.at[0]],
                              ssems[b]).wait()

    def half(h, carry):
        pltpu.sync_copy(src_hbm.at[s, h], src_v)
        pltpu.sync_copy(dst_hbm.at[s, h], dst_v)

        def rmr(j, carry2):
            def rmc(k, carry3):
                v = dst_v[j, pl.ds(k * 16, 16)]
                lv = v - lo
                ok = (lv >= 0) & (lv < NPC)
                dst_v[j, pl.ds(k * 16, 16)] = jnp.where(
                    ok, lv, jnp.full((16,), NPC, jnp.int32))
                return carry3
            return lax.fori_loop(0, CH // 16, rmc, carry2)
        lax.fori_loop(0, NCH_H, rmr, 0)

        for b in range(3):
            pltpu.async_copy(g_hbm.at[src_v.at[b]], rows_v.at[b], gsems[b])

        def body(jj, carry2):
            for b in range(4):
                j = jj * 4 + b
                _gwait(b)
                pltpu.async_copy(rows_v.at[b], acc_sh.at[dst_v.at[j]],
                                 ssems[b], add=True)
                bn = (b + 3) % 4
                if b == 0:
                    @pl.when(jj > 0)
                    def _():
                        _swait(bn)
                else:
                    _swait(bn)
                if b == 0:
                    pltpu.async_copy(g_hbm.at[src_v.at[j + 3]],
                                     rows_v.at[bn], gsems[bn])
                else:
                    @pl.when(jj < NCH_H // 4 - 1)
                    def _():
                        pltpu.async_copy(g_hbm.at[src_v.at[j + 3]],
                                         rows_v.at[bn], gsems[bn])
            return carry2
        lax.fori_loop(0, NCH_H // 4, body, 0)
        _swait(3)
        return carry

    lax.fori_loop(0, NHALF, half, 0)
    plsc.subcore_barrier()

    def out_chunk(k, carry):
        base = s * ORPT + k * OCH
        @pl.when(base < NPC)
        def _():
            pltpu.sync_copy(acc_sh.at[pl.ds(base, OCH)],
                            rows_v.at[0].at[pl.ds(0, OCH)])
            pltpu.sync_copy(rows_v.at[0].at[pl.ds(0, OCH)],
                            out_hbm.at[pl.ds(c * NPC + base, OCH)])
        return carry
    lax.fori_loop(0, ORPT // OCH, out_chunk, 0)


BLK = 2000
GRID = N // BLK


def _tc_first_body(cnt_ref, x_ref, w_ref, g_ref, dis_ref):
    deg = 1.0 + cnt_ref[0] + cnt_ref[1]
    dis = lax.rsqrt(deg)
    y = jnp.dot(x_ref[...], w_ref[...], preferred_element_type=jnp.float32)
    g_ref[...] = y * dis
    dis_ref[...] = dis


def _tc_first(cnt, x, w):
    return pl.pallas_call(
        _tc_first_body,
        grid=(GRID,),
        in_specs=[
            pl.BlockSpec((NC, BLK, 1), lambda i: (0, i, 0)),
            pl.BlockSpec((BLK, D), lambda i: (i, 0)),
            pl.BlockSpec((D, D), lambda i: (0, 0)),
        ],
        out_specs=[
            pl.BlockSpec((BLK, D), lambda i: (i, 0)),
            pl.BlockSpec((BLK, 1), lambda i: (i, 0)),
        ],
        out_shape=[
            jax.ShapeDtypeStruct((N, D), jnp.float32),
            jax.ShapeDtypeStruct((N, 1), jnp.float32),
        ],
    )(cnt, x, w)


def _tc_mid_body(p_ref, g_ref, dis_ref, b_ref, w_ref, gn_ref):
    p = p_ref[...] + g_ref[...]
    h = jnp.maximum(dis_ref[...] * p + b_ref[...], 0.0)
    y = jnp.dot(h, w_ref[...], preferred_element_type=jnp.float32)
    gn_ref[...] = y * dis_ref[...]


def _tc_mid(p, g, dis, b_prev, w_next):
    return pl.pallas_call(
        _tc_mid_body,
        grid=(GRID,),
        in_specs=[
            pl.BlockSpec((BLK, D), lambda i: (i, 0)),
            pl.BlockSpec((BLK, D), lambda i: (i, 0)),
            pl.BlockSpec((BLK, 1), lambda i: (i, 0)),
            pl.BlockSpec((1, D), lambda i: (0, 0)),
            pl.BlockSpec((D, D), lambda i: (0, 0)),
        ],
        out_specs=pl.BlockSpec((BLK, D), lambda i: (i, 0)),
        out_shape=jax.ShapeDtypeStruct((N, D), jnp.float32),
    )(p, g, dis, b_prev, w_next)


def _tc_final_body(p_ref, g_ref, dis_ref, b_ref, out_ref):
    p = p_ref[...] + g_ref[...]
    out_ref[...] = dis_ref[...] * p + b_ref[...]


def _tc_final(p, g, dis, b):
    return pl.pallas_call(
        _tc_final_body,
        grid=(GRID,),
        in_specs=[
            pl.BlockSpec((BLK, D), lambda i: (i, 0)),
            pl.BlockSpec((BLK, D), lambda i: (i, 0)),
            pl.BlockSpec((BLK, 1), lambda i: (i, 0)),
            pl.BlockSpec((1, D), lambda i: (0, 0)),
        ],
        out_specs=pl.BlockSpec((BLK, D), lambda i: (i, 0)),
        out_shape=jax.ShapeDtypeStruct((N, D), jnp.float32),
    )(p, g, dis, b)


def kernel(x, edge_index, W0, b0, W1, b1, W2, b2):
    pad = EPAD - E
    srcf = jnp.concatenate([edge_index[0], jnp.zeros((pad,), jnp.int32)])
    dstf = jnp.concatenate([edge_index[1], jnp.full((pad,), N, jnp.int32)])
    src = srcf.reshape(NS, NHALF, NCH_H, CH)
    dst = dstf.reshape(NS, NHALF, NCH_H, CH)
    dstw = edge_index[1].reshape(NW, DCHUNK, CH)
    b0r = b0.reshape(1, D)
    b1r = b1.reshape(1, D)
    b2r = b2.reshape(1, D)

    cnt = _sc_degree(dstw).reshape(NC, N, 1)
    g0, dis = _tc_first(cnt, x, W0)
    p0 = _sc_scatter(g0, src, dst)
    g1 = _tc_mid(p0, g0, dis, b0r, W1)
    p1 = _sc_scatter(g1, src, dst)
    g2 = _tc_mid(p1, g1, dis, b1r, W2)
    p2 = _sc_scatter(g2, src, dst)
    return _tc_final(p2, g2, dis, b2r)


# speedup vs baseline: 1.7911x; 1.7911x over previous
import functools

import jax
import jax.numpy as jnp
from jax import lax
from jax.experimental import pallas as pl
from jax.experimental.pallas import tpu as pltpu
from jax.experimental.pallas import tpu_sc as plsc

N = 10000
E = 320000
D = 128
NC = 2
NS = 16
NPC = N // NC
ACC = 5120
CH = 80
NCH_H = 126
NHALF = 2
EPT = NHALF * NCH_H * CH
EPAD = NS * EPT
NW = NC * NS
EPW = E // NW
DCHUNK = EPW // CH
ZRPT = ACC // NS
RCH = 80
OCH = 40
ORPT = 320

_mesh = plsc.VectorSubcoreMesh(core_axis_name="c", subcore_axis_name="s")


@functools.partial(
    pl.kernel,
    out_type=jax.ShapeDtypeStruct((NC * N,), jnp.float32),
    mesh=_mesh,
    scratch_types=[
        pltpu.VMEM((DCHUNK, CH), jnp.int32),
        pltpu.VMEM((CH,), jnp.float32),
        pltpu.VMEM((2000,), jnp.float32),
        pltpu.VMEM_SHARED((N,), jnp.float32),
        pltpu.SemaphoreType.DMA,
    ],
)
def _sc_degree(dst_hbm, cnt_hbm, dst_v, ones_v, zbuf_v, deg_sh, sem):
    c = lax.axis_index("c")
    s = lax.axis_index("s")
    wid = c * NS + s

    def zb(i, carry):
        zbuf_v[pl.ds(i * 16, 16)] = jnp.zeros((16,), jnp.float32)
        return carry
    lax.fori_loop(0, 125, zb, 0)
    @pl.when(s < 5)
    def _zero_acc():
        pltpu.sync_copy(zbuf_v, deg_sh.at[pl.ds(s * 2000, 2000)])

    for k in range(CH // 16):
        ones_v[pl.ds(k * 16, 16)] = jnp.ones((16,), jnp.float32)
    pltpu.sync_copy(dst_hbm.at[wid], dst_v)
    plsc.subcore_barrier()

    def body(j, carry):
        pltpu.async_copy(ones_v, deg_sh.at[dst_v.at[j]], sem, add=True).wait()
        return carry
    lax.fori_loop(0, DCHUNK, body, 0)
    plsc.subcore_barrier()

    @pl.when(s < 5)
    def _copy_out():
        pltpu.sync_copy(deg_sh.at[pl.ds(s * 2000, 2000)], zbuf_v)
        pltpu.sync_copy(zbuf_v, cnt_hbm.at[pl.ds(c * N + s * 2000, 2000)])


@functools.partial(
    pl.kernel,
    out_type=jax.ShapeDtypeStruct((N, D), jnp.float32),
    mesh=_mesh,
    scratch_types=[
        pltpu.VMEM((NCH_H, CH), jnp.int32),
        pltpu.VMEM((NCH_H, CH), jnp.int32),
        pltpu.VMEM((3, CH, D), jnp.float32),
        pltpu.VMEM_SHARED((ACC, D), jnp.float32),
        pltpu.SemaphoreType.DMA,
        pltpu.SemaphoreType.DMA,
        pltpu.SemaphoreType.DMA,
        pltpu.SemaphoreType.DMA,
    ],
)
def _sc_scatter(g_hbm, src_hbm, dst_hbm, out_hbm,
                src_v, dst_v, rows_v, acc_sh,
                gsem0, gsem1, gsem2, ssem):
    c = lax.axis_index("c")
    s = lax.axis_index("s")

    def zr(r, carry):
        def zc(k, carry2):
            rows_v[0, r, pl.ds(k * 16, 16)] = jnp.zeros((16,), jnp.float32)
            return carry2
        return lax.fori_loop(0, D // 16, zc, carry)
    lax.fori_loop(0, RCH, zr, 0)

    def zchunk(k, carry):
        pltpu.sync_copy(rows_v.at[0], acc_sh.at[pl.ds(s * ZRPT + k * RCH, RCH)])
        return carry
    lax.fori_loop(0, ZRPT // RCH, zchunk, 0)

    gsems = (gsem0, gsem1, gsem2)
    lo = c * NPC

    def _gwait(b):
        pltpu.make_async_copy(g_hbm.at[src_v.at[0]], rows_v.at[b],
                              gsems[b]).wait()

    def _swait():
        pltpu.make_async_copy(rows_v.at[0], acc_sh.at[dst_v.at[0]],
                              ssem).wait()

    def half(h, carry):
        pltpu.sync_copy(src_hbm.at[s, h], src_v)
        pltpu.sync_copy(dst_hbm.at[s, h], dst_v)

        def rmr(j, carry2):
            def rmc(k, carry3):
                v = dst_v[j, pl.ds(k * 16, 16)]
                lv = v - lo
                ok = (lv >= 0) & (lv < NPC)
                dst_v[j, pl.ds(k * 16, 16)] = jnp.where(
                    ok, lv, jnp.full((16,), NPC, jnp.int32))
                return carry3
            return lax.fori_loop(0, CH // 16, rmc, carry2)
        lax.fori_loop(0, NCH_H, rmr, 0)

        for b in range(2):
            pltpu.async_copy(g_hbm.at[src_v.at[b]], rows_v.at[b], gsems[b])

        def body(jj, carry2):
            for b in range(3):
                j = jj * 3 + b
                bn = (b + 2) % 3
                _gwait(b)
                pltpu.async_copy(rows_v.at[b], acc_sh.at[dst_v.at[j]],
                                 ssem, add=True)
                if b == 0:
                    pltpu.async_copy(g_hbm.at[src_v.at[j + 2]],
                                     rows_v.at[bn], gsems[bn])
                else:
                    @pl.when(jj < NCH_H // 3 - 1)
                    def _():
                        pltpu.async_copy(g_hbm.at[src_v.at[j + 2]],
                                         rows_v.at[bn], gsems[bn])
                _swait()
            return carry2
        lax.fori_loop(0, NCH_H // 3, body, 0)
        return carry

    lax.fori_loop(0, NHALF, half, 0)
    plsc.subcore_barrier()

    def out_chunk(k, carry):
        base = s * ORPT + k * OCH
        @pl.when(base < NPC)
        def _():
            pltpu.sync_copy(acc_sh.at[pl.ds(base, OCH)],
                            rows_v.at[0].at[pl.ds(0, OCH)])
            pltpu.sync_copy(rows_v.at[0].at[pl.ds(0, OCH)],
                            out_hbm.at[pl.ds(c * NPC + base, OCH)])
        return carry
    lax.fori_loop(0, ORPT // OCH, out_chunk, 0)


BLK = 2000
GRID = N // BLK


def _tc_first_body(cnt_ref, x_ref, w_ref, g_ref, dis_ref):
    deg = 1.0 + cnt_ref[0] + cnt_ref[1]
    dis = lax.rsqrt(deg)
    y = jnp.dot(x_ref[...], w_ref[...], preferred_element_type=jnp.float32)
    g_ref[...] = y * dis
    dis_ref[...] = dis


def _tc_first(cnt, x, w):
    return pl.pallas_call(
        _tc_first_body,
        grid=(GRID,),
        in_specs=[
            pl.BlockSpec((NC, BLK, 1), lambda i: (0, i, 0)),
            pl.BlockSpec((BLK, D), lambda i: (i, 0)),
            pl.BlockSpec((D, D), lambda i: (0, 0)),
        ],
        out_specs=[
            pl.BlockSpec((BLK, D), lambda i: (i, 0)),
            pl.BlockSpec((BLK, 1), lambda i: (i, 0)),
        ],
        out_shape=[
            jax.ShapeDtypeStruct((N, D), jnp.float32),
            jax.ShapeDtypeStruct((N, 1), jnp.float32),
        ],
    )(cnt, x, w)


def _tc_mid_body(p_ref, g_ref, dis_ref, b_ref, w_ref, gn_ref):
    p = p_ref[...] + g_ref[...]
    h = jnp.maximum(dis_ref[...] * p + b_ref[...], 0.0)
    y = jnp.dot(h, w_ref[...], preferred_element_type=jnp.float32)
    gn_ref[...] = y * dis_ref[...]


def _tc_mid(p, g, dis, b_prev, w_next):
    return pl.pallas_call(
        _tc_mid_body,
        grid=(GRID,),
        in_specs=[
            pl.BlockSpec((BLK, D), lambda i: (i, 0)),
            pl.BlockSpec((BLK, D), lambda i: (i, 0)),
            pl.BlockSpec((BLK, 1), lambda i: (i, 0)),
            pl.BlockSpec((1, D), lambda i: (0, 0)),
            pl.BlockSpec((D, D), lambda i: (0, 0)),
        ],
        out_specs=pl.BlockSpec((BLK, D), lambda i: (i, 0)),
        out_shape=jax.ShapeDtypeStruct((N, D), jnp.float32),
    )(p, g, dis, b_prev, w_next)


def _tc_final_body(p_ref, g_ref, dis_ref, b_ref, out_ref):
    p = p_ref[...] + g_ref[...]
    out_ref[...] = dis_ref[...] * p + b_ref[...]


def _tc_final(p, g, dis, b):
    return pl.pallas_call(
        _tc_final_body,
        grid=(GRID,),
        in_specs=[
            pl.BlockSpec((BLK, D), lambda i: (i, 0)),
            pl.BlockSpec((BLK, D), lambda i: (i, 0)),
            pl.BlockSpec((BLK, 1), lambda i: (i, 0)),
            pl.BlockSpec((1, D), lambda i: (0, 0)),
        ],
        out_specs=pl.BlockSpec((BLK, D), lambda i: (i, 0)),
        out_shape=jax.ShapeDtypeStruct((N, D), jnp.float32),
    )(p, g, dis, b)


def kernel(x, edge_index, W0, b0, W1, b1, W2, b2):
    pad = EPAD - E
    srcf = jnp.concatenate([edge_index[0], jnp.zeros((pad,), jnp.int32)])
    dstf = jnp.concatenate([edge_index[1], jnp.full((pad,), N, jnp.int32)])
    src = srcf.reshape(NS, NHALF, NCH_H, CH)
    dst = dstf.reshape(NS, NHALF, NCH_H, CH)
    dstw = edge_index[1].reshape(NW, DCHUNK, CH)
    b0r = b0.reshape(1, D)
    b1r = b1.reshape(1, D)
    b2r = b2.reshape(1, D)

    cnt = _sc_degree(dstw).reshape(NC, N, 1)
    g0, dis = _tc_first(cnt, x, W0)
    p0 = _sc_scatter(g0, src, dst)
    g1 = _tc_mid(p0, g0, dis, b0r, W1)
    p1 = _sc_scatter(g1, src, dst)
    g2 = _tc_mid(p1, g1, dis, b1r, W2)
    p2 = _sc_scatter(g2, src, dst)
    return _tc_final(p2, g2, dis, b2r)


# speedup vs baseline: 2.9002x; 1.6193x over previous
import functools

import jax
import jax.numpy as jnp
from jax import lax
from jax.experimental import pallas as pl
from jax.experimental.pallas import tpu as pltpu
from jax.experimental.pallas import tpu_sc as plsc

N = 10000
E = 320000
D = 128
NC = 2
NS = 16
NPC = N // NC
ACC = 5120
CH = 80
EPT = E // NS
NCHUNK = EPT // CH
NW = NC * NS
EPW = E // NW
DCHUNK = EPW // CH
ZRPT = ACC // NS
RCH = 80
OCH = 40
ORPT = 320

_mesh = plsc.VectorSubcoreMesh(core_axis_name="c", subcore_axis_name="s")


@functools.partial(
    pl.kernel,
    out_type=jax.ShapeDtypeStruct((NC * N,), jnp.float32),
    mesh=_mesh,
    scratch_types=[
        pltpu.VMEM((DCHUNK, CH), jnp.int32),
        pltpu.VMEM((CH,), jnp.float32),
        pltpu.VMEM((2000,), jnp.float32),
        pltpu.VMEM_SHARED((N,), jnp.float32),
        pltpu.SemaphoreType.DMA,
    ],
)
def _sc_degree(dst_hbm, cnt_hbm, dst_v, ones_v, zbuf_v, deg_sh, sem):
    c = lax.axis_index("c")
    s = lax.axis_index("s")
    wid = c * NS + s

    def zb(i, carry):
        zbuf_v[pl.ds(i * 16, 16)] = jnp.zeros((16,), jnp.float32)
        return carry
    lax.fori_loop(0, 125, zb, 0)
    @pl.when(s < 5)
    def _zero_acc():
        pltpu.sync_copy(zbuf_v, deg_sh.at[pl.ds(s * 2000, 2000)])

    for k in range(CH // 16):
        ones_v[pl.ds(k * 16, 16)] = jnp.ones((16,), jnp.float32)
    pltpu.sync_copy(dst_hbm.at[wid], dst_v)
    plsc.subcore_barrier()

    def body(j, carry):
        pltpu.async_copy(ones_v, deg_sh.at[dst_v.at[j]], sem, add=True).wait()
        return carry
    lax.fori_loop(0, DCHUNK, body, 0)
    plsc.subcore_barrier()

    @pl.when(s < 5)
    def _copy_out():
        pltpu.sync_copy(deg_sh.at[pl.ds(s * 2000, 2000)], zbuf_v)
        pltpu.sync_copy(zbuf_v, cnt_hbm.at[pl.ds(c * N + s * 2000, 2000)])


@functools.partial(
    pl.kernel,
    out_type=jax.ShapeDtypeStruct((N, D), jnp.float32),
    mesh=_mesh,
    scratch_types=[
        pltpu.VMEM((NCHUNK, CH), jnp.int32),
        pltpu.VMEM((NCHUNK, CH), jnp.int32),
        pltpu.VMEM((2, CH, D), jnp.float32),
        pltpu.VMEM_SHARED((ACC, D), jnp.float32),
        pltpu.SemaphoreType.DMA,
        pltpu.SemaphoreType.DMA,
        pltpu.SemaphoreType.DMA,
        pltpu.SemaphoreType.DMA,
    ],
)
def _sc_scatter(g_hbm, src_hbm, dst_hbm, out_hbm,
                src_v, dst_v, rows_v, acc_sh,
                gsem0, gsem1, ssem0, ssem1):
    c = lax.axis_index("c")
    s = lax.axis_index("s")

    def zr(r, carry):
        def zc(k, carry2):
            rows_v[0, r, pl.ds(k * 16, 16)] = jnp.zeros((16,), jnp.float32)
            return carry2
        return lax.fori_loop(0, D // 16, zc, carry)
    lax.fori_loop(0, RCH, zr, 0)

    def zchunk(k, carry):
        pltpu.sync_copy(rows_v.at[0], acc_sh.at[pl.ds(s * ZRPT + k * RCH, RCH)])
        return carry
    lax.fori_loop(0, ZRPT // RCH, zchunk, 0)

    pltpu.sync_copy(src_hbm.at[s], src_v)
    pltpu.sync_copy(dst_hbm.at[s], dst_v)

    lo = c * NPC
    def rmr(j, carry):
        def rmc(k, carry2):
            v = dst_v[j, pl.ds(k * 16, 16)]
            lv = v - lo
            ok = (lv >= 0) & (lv < NPC)
            dst_v[j, pl.ds(k * 16, 16)] = jnp.where(
                ok, lv, jnp.full((16,), NPC, jnp.int32))
            return carry2
        return lax.fori_loop(0, CH // 16, rmc, carry)
    lax.fori_loop(0, NCHUNK, rmr, 0)
    plsc.subcore_barrier()

    def _gwait(b, sem):
        pltpu.make_async_copy(g_hbm.at[src_v.at[0]], rows_v.at[b], sem).wait()

    def _swait(b, sem):
        pltpu.make_async_copy(rows_v.at[b], acc_sh.at[dst_v.at[0]], sem).wait()

    pltpu.async_copy(g_hbm.at[src_v.at[0]], rows_v.at[0], gsem0)

    def body(jj, carry):
        j0 = jj * 2
        j1 = j0 + 1
        @pl.when(jj > 0)
        def _():
            _swait(1, ssem1)
        pltpu.async_copy(g_hbm.at[src_v.at[j1]], rows_v.at[1], gsem1)
        _gwait(0, gsem0)
        pltpu.async_copy(rows_v.at[0], acc_sh.at[dst_v.at[j0]], ssem0,
                         add=True)
        _swait(0, ssem0)
        @pl.when(jj < NCHUNK // 2 - 1)
        def _():
            pltpu.async_copy(g_hbm.at[src_v.at[j0 + 2]], rows_v.at[0], gsem0)
        _gwait(1, gsem1)
        pltpu.async_copy(rows_v.at[1], acc_sh.at[dst_v.at[j1]], ssem1,
                         add=True)
        return carry
    lax.fori_loop(0, NCHUNK // 2, body, 0)
    _swait(1, ssem1)
    plsc.subcore_barrier()

    def out_chunk(k, carry):
        base = s * ORPT + k * OCH
        @pl.when(base < NPC)
        def _():
            pltpu.sync_copy(acc_sh.at[pl.ds(base, OCH)],
                            rows_v.at[0].at[pl.ds(0, OCH)])
            pltpu.sync_copy(rows_v.at[0].at[pl.ds(0, OCH)],
                            out_hbm.at[pl.ds(c * NPC + base, OCH)])
        return carry
    lax.fori_loop(0, ORPT // OCH, out_chunk, 0)


BLK = 2000
GRID = N // BLK


def _tc_first_body(cnt_ref, x_ref, w_ref, g_ref, dis_ref):
    deg = 1.0 + cnt_ref[0] + cnt_ref[1]
    dis = lax.rsqrt(deg)
    y = jnp.dot(x_ref[...], w_ref[...], preferred_element_type=jnp.float32)
    g_ref[...] = y * dis
    dis_ref[...] = dis


def _tc_first(cnt, x, w):
    return pl.pallas_call(
        _tc_first_body,
        grid=(GRID,),
        in_specs=[
            pl.BlockSpec((NC, BLK, 1), lambda i: (0, i, 0)),
            pl.BlockSpec((BLK, D), lambda i: (i, 0)),
            pl.BlockSpec((D, D), lambda i: (0, 0)),
        ],
        out_specs=[
            pl.BlockSpec((BLK, D), lambda i: (i, 0)),
            pl.BlockSpec((BLK, 1), lambda i: (i, 0)),
        ],
        out_shape=[
            jax.ShapeDtypeStruct((N, D), jnp.float32),
            jax.ShapeDtypeStruct((N, 1), jnp.float32),
        ],
    )(cnt, x, w)


def _tc_mid_body(p_ref, g_ref, dis_ref, b_ref, w_ref, gn_ref):
    p = p_ref[...] + g_ref[...]
    h = jnp.maximum(dis_ref[...] * p + b_ref[...], 0.0)
    y = jnp.dot(h, w_ref[...], preferred_element_type=jnp.float32)
    gn_ref[...] = y * dis_ref[...]


def _tc_mid(p, g, dis, b_prev, w_next):
    return pl.pallas_call(
        _tc_mid_body,
        grid=(GRID,),
        in_specs=[
            pl.BlockSpec((BLK, D), lambda i: (i, 0)),
            pl.BlockSpec((BLK, D), lambda i: (i, 0)),
            pl.BlockSpec((BLK, 1), lambda i: (i, 0)),
            pl.BlockSpec((1, D), lambda i: (0, 0)),
            pl.BlockSpec((D, D), lambda i: (0, 0)),
        ],
        out_specs=pl.BlockSpec((BLK, D), lambda i: (i, 0)),
        out_shape=jax.ShapeDtypeStruct((N, D), jnp.float32),
    )(p, g, dis, b_prev, w_next)


def _tc_final_body(p_ref, g_ref, dis_ref, b_ref, out_ref):
    p = p_ref[...] + g_ref[...]
    out_ref[...] = dis_ref[...] * p + b_ref[...]


def _tc_final(p, g, dis, b):
    return pl.pallas_call(
        _tc_final_body,
        grid=(GRID,),
        in_specs=[
            pl.BlockSpec((BLK, D), lambda i: (i, 0)),
            pl.BlockSpec((BLK, D), lambda i: (i, 0)),
            pl.BlockSpec((BLK, 1), lambda i: (i, 0)),
            pl.BlockSpec((1, D), lambda i: (0, 0)),
        ],
        out_specs=pl.BlockSpec((BLK, D), lambda i: (i, 0)),
        out_shape=jax.ShapeDtypeStruct((N, D), jnp.float32),
    )(p, g, dis, b)


def kernel(x, edge_index, W0, b0, W1, b1, W2, b2):
    src = edge_index[0].reshape(NS, NCHUNK, CH)
    dst = edge_index[1].reshape(NS, NCHUNK, CH)
    dstw = edge_index[1].reshape(NW, DCHUNK, CH)
    b0r = b0.reshape(1, D)
    b1r = b1.reshape(1, D)
    b2r = b2.reshape(1, D)

    cnt = _sc_degree(dstw).reshape(NC, N, 1)
    g0, dis = _tc_first(cnt, x, W0)
    p0 = _sc_scatter(g0, src, dst)
    g1 = _tc_mid(p0, g0, dis, b0r, W1)
    p1 = _sc_scatter(g1, src, dst)
    g2 = _tc_mid(p1, g1, dis, b1r, W2)
    p2 = _sc_scatter(g2, src, dst)
    return _tc_final(p2, g2, dis, b2r)
